# baseline (speedup 1.0000x reference)
import functools

import jax
import jax.numpy as jnp
from jax import lax
from jax.experimental import pallas as pl
from jax.experimental.pallas import tpu as pltpu
from jax.experimental.pallas import tpu_sc as plsc

_C = 10
_N = 1048576
_LANES = 16
_NW = 32
_RW = _N // _NW
_R = 4096
_NCHUNK = _RW // _R
_G = _R // _LANES
_BINS = 128
_MINOR = 640
_RPC = _R * _C // _MINOR

_mesh = plsc.VectorSubcoreMesh(core_axis_name="c", subcore_axis_name="s")


@functools.partial(
    pl.kernel,
    out_type=jax.ShapeDtypeStruct((_NW, _BINS), jnp.int32),
    mesh=_mesh,
    compiler_params=pltpu.CompilerParams(needs_layout_passes=False),
    scratch_types=[
        pltpu.VMEM((_RPC, _MINOR), jnp.float32),
        pltpu.VMEM((_RPC, _MINOR), jnp.float32),
        pltpu.VMEM((_R,), jnp.int32),
        pltpu.VMEM((_R,), jnp.int32),
        pltpu.VMEM((_LANES * _BINS,), jnp.int32),
        pltpu.SemaphoreType.DMA,
        pltpu.SemaphoreType.DMA,
        pltpu.SemaphoreType.DMA,
        pltpu.SemaphoreType.DMA,
    ],
)
def _confusion(yp_hbm, yt_hbm, out_hbm, ybuf0, ybuf1, tbuf0, tbuf1, cmbuf,
               sp0, sp1, st0, st1):
    wid = lax.axis_index("s") * 2 + lax.axis_index("c")
    base = wid * _RW
    ybufs = (ybuf0, ybuf1)
    tbufs = (tbuf0, tbuf1)
    sems_p = (sp0, sp1)
    sems_t = (st0, st1)

    def start(i):
        b = i % 2
        off = base + i * _R
        vrow = pl.multiple_of(wid * (_RW * _C // _MINOR) + i * _RPC, 8)
        cp = pltpu.make_async_copy(
            yp_hbm.at[pl.ds(vrow, _RPC)], ybufs[b], sems_p[b])
        cp.start()
        ct = pltpu.make_async_copy(
            yt_hbm.at[pl.ds(off, _R)], tbufs[b], sems_t[b])
        ct.start()
        return cp, ct

    handles = {0: start(0), 1: start(1)}

    iota = lax.iota(jnp.int32, _LANES)
    zero = jnp.zeros((_LANES,), jnp.int32)
    ones = jnp.ones((_LANES,), jnp.int32)
    lane_off = iota * _BINS

    for j in range(_BINS):
        cmbuf[pl.ds(j * _LANES, _LANES)] = zero

    for i in range(_NCHUNK):
        b = i % 2
        for h in handles.pop(i):
            h.wait()
        ybuf_b = ybufs[b]
        tbuf_b = tbufs[b]

        def body(g, carry, ybuf_b=ybuf_b, tbuf_b=tbuf_b):
            rowb = g * _LANES + iota
            t = plsc.load_gather(tbuf_b, [rowb])
            vr = lax.shift_right_logical(rowb, 6)
            vcb = (rowb & 63) * _C
            cands = [(plsc.load_gather(ybuf_b, [vr, vcb + c]),
                      jnp.full((_LANES,), c, jnp.int32)) for c in range(_C)]
            while len(cands) > 1:
                nxt = []
                for k in range(0, len(cands) - 1, 2):
                    (va, pa), (vb, pb) = cands[k], cands[k + 1]
                    gt = vb > va
                    nxt.append((jnp.where(gt, vb, va), jnp.where(gt, pb, pa)))
                if len(cands) % 2:
                    nxt.append(cands[-1])
                cands = nxt
            p = cands[0][1]
            plsc.addupdate_scatter(cmbuf, [lane_off + (t * _C + p)], ones)
            return carry

        lax.fori_loop(0, _G, body, 0, unroll=4)
        if i + 2 < _NCHUNK:
            handles[i + 2] = start(i + 2)

    half = _LANES // 2
    while half >= 1:
        for l in range(half):
            for j in range(_BINS // _LANES):
                a = l * _BINS + j * _LANES
                bb = (l + half) * _BINS + j * _LANES
                cmbuf[pl.ds(a, _LANES)] = (
                    cmbuf[pl.ds(a, _LANES)] + cmbuf[pl.ds(bb, _LANES)])
        half //= 2

    pltpu.sync_copy(cmbuf.at[pl.ds(0, _BINS)], out_hbm.at[wid])


def kernel(y_pred, y_true):
    yp = y_pred.reshape(_N * _C // _MINOR, _MINOR)
    yt = y_true.reshape(-1).astype(jnp.int32)
    parts = _confusion(yp, yt)
    counts = parts.sum(axis=0)[: _C * _C].reshape(_C, _C)
    cm = counts.astype(jnp.float32)
    hist_true = cm.sum(axis=1)
    hist_pred = cm.sum(axis=0)
    cmn = cm / cm.sum()
    expected = jnp.outer(hist_true, hist_pred)
    expected = expected / expected.sum()
    i = jnp.arange(_C, dtype=jnp.float32)
    weight_matrix = (i[:, None] - i[None, :]) ** 2
    return 1.0 - (weight_matrix * cmn).sum() / (weight_matrix * expected).sum()


# speedup vs baseline: 1.1142x; 1.1142x over previous
import functools

import jax
import jax.numpy as jnp
from jax import lax
from jax.experimental import pallas as pl
from jax.experimental.pallas import tpu as pltpu
from jax.experimental.pallas import tpu_sc as plsc

_C = 10
_N = 1048576
_LANES = 16
_NW = 32
_RW = _N // _NW
_R = 2048
_NCHUNK = _RW // _R
_G = _R // _LANES
_BINS = 128
_MINOR = 80
_RPC = _R * _C // _MINOR

_mesh = plsc.VectorSubcoreMesh(core_axis_name="c", subcore_axis_name="s")


@functools.partial(
    pl.kernel,
    out_type=jax.ShapeDtypeStruct((_NW, _BINS), jnp.int32),
    mesh=_mesh,
    compiler_params=pltpu.CompilerParams(
        needs_layout_passes=False, use_tc_tiling_on_sc=True),
    scratch_types=[
        pltpu.VMEM((_RPC, _MINOR), jnp.float32),
        pltpu.VMEM((_RPC, _MINOR), jnp.float32),
        pltpu.VMEM((_R,), jnp.int32),
        pltpu.VMEM((_R,), jnp.int32),
        pltpu.VMEM((_LANES * _BINS,), jnp.int32),
        pltpu.SemaphoreType.DMA,
        pltpu.SemaphoreType.DMA,
        pltpu.SemaphoreType.DMA,
        pltpu.SemaphoreType.DMA,
    ],
)
def _confusion(yp_hbm, yt_hbm, out_hbm, ybuf0, ybuf1, tbuf0, tbuf1, cmbuf,
               sp0, sp1, st0, st1):
    wid = lax.axis_index("s") * 2 + lax.axis_index("c")
    base = wid * _RW
    ybufs = (ybuf0, ybuf1)
    tbufs = (tbuf0, tbuf1)
    sems_p = (sp0, sp1)
    sems_t = (st0, st1)

    def start(i):
        b = i % 2
        off = base + i * _R
        vrow = pl.multiple_of(wid * (_RW * _C // _MINOR) + i * _RPC, 8)
        cp = pltpu.make_async_copy(
            yp_hbm.at[pl.ds(vrow, _RPC)], ybufs[b], sems_p[b])
        cp.start()
        ct = pltpu.make_async_copy(
            yt_hbm.at[pl.ds(off, _R)], tbufs[b], sems_t[b])
        ct.start()
        return cp, ct

    handles = {0: start(0), 1: start(1)}

    iota = lax.iota(jnp.int32, _LANES)
    zero = jnp.zeros((_LANES,), jnp.int32)
    ones = jnp.ones((_LANES,), jnp.int32)
    lane_off = iota * _BINS

    for j in range(_BINS):
        cmbuf[pl.ds(j * _LANES, _LANES)] = zero

    for i in range(_NCHUNK):
        b = i % 2
        for h in handles.pop(i):
            h.wait()
        ybuf_b = ybufs[b]
        tbuf_b = tbufs[b]

        def body(g, carry, ybuf_b=ybuf_b, tbuf_b=tbuf_b):
            rowb = g * _LANES + iota
            t = plsc.load_gather(tbuf_b, [rowb])
            vr = lax.shift_right_logical(rowb, 3)
            vcb = (rowb & 7) * _C
            cands = [(plsc.load_gather(ybuf_b, [vr, vcb + c]),
                      jnp.full((_LANES,), c, jnp.int32)) for c in range(_C)]
            while len(cands) > 1:
                nxt = []
                for k in range(0, len(cands) - 1, 2):
                    (va, pa), (vb, pb) = cands[k], cands[k + 1]
                    gt = vb > va
                    nxt.append((jnp.where(gt, vb, va), jnp.where(gt, pb, pa)))
                if len(cands) % 2:
                    nxt.append(cands[-1])
                cands = nxt
            p = cands[0][1]
            plsc.addupdate_scatter(cmbuf, [lane_off + (t * _C + p)], ones)
            return carry

        lax.fori_loop(0, _G, body, 0, unroll=4)
        if i + 2 < _NCHUNK:
            handles[i + 2] = start(i + 2)

    half = _LANES // 2
    while half >= 1:
        for l in range(half):
            for j in range(_BINS // _LANES):
                a = l * _BINS + j * _LANES
                bb = (l + half) * _BINS + j * _LANES
                cmbuf[pl.ds(a, _LANES)] = (
                    cmbuf[pl.ds(a, _LANES)] + cmbuf[pl.ds(bb, _LANES)])
        half //= 2

    pltpu.sync_copy(cmbuf.at[pl.ds(0, _BINS)], out_hbm.at[wid])


def kernel(y_pred, y_true):
    yp = y_pred.reshape(_N * _C // _MINOR, _MINOR)
    yt = y_true.reshape(-1).astype(jnp.int32)
    parts = _confusion(yp, yt)
    counts = parts.sum(axis=0)[: _C * _C].reshape(_C, _C)
    cm = counts.astype(jnp.float32)
    hist_true = cm.sum(axis=1)
    hist_pred = cm.sum(axis=0)
    cmn = cm / cm.sum()
    expected = jnp.outer(hist_true, hist_pred)
    expected = expected / expected.sum()
    i = jnp.arange(_C, dtype=jnp.float32)
    weight_matrix = (i[:, None] - i[None, :]) ** 2
    return 1.0 - (weight_matrix * cmn).sum() / (weight_matrix * expected).sum()
